# Initial kernel scaffold; baseline (speedup 1.0000x reference)
import jax
import jax.numpy as jnp
from jax.experimental import pallas as pl


def kernel(xs):
    raise NotImplementedError("write your pallas kernel here")


# speedup vs baseline: 2.2759x; 2.2759x over previous
import jax
import jax.numpy as jnp
from jax import lax
from jax.experimental import pallas as pl
from jax.experimental.pallas import tpu as pltpu
from jax.experimental.pallas import tpu_sc as plsc

ROWS = 128
N = 32768
L = 16
C = N // L
BINS = 256
NC, NS = 2, 16
NW = NC * NS
RPW = ROWS // NW

import numpy as np

_MININT = np.int32(-(2 ** 31))
_ALLONES = np.int32(-1)


def _digit(k, p):
    if p == 0:
        s = k
    else:
        s = lax.shift_right_logical(k, jnp.full((L,), 8 * p, jnp.int32))
    return jnp.bitwise_and(s, jnp.full((L,), 255, jnp.int32))


def _to_key(b):
    return jnp.where(b < 0, jnp.bitwise_xor(b, _ALLONES),
                     jnp.bitwise_xor(b, _MININT))


def _from_key(k):
    return jnp.where(k < 0, jnp.bitwise_xor(k, _MININT),
                     jnp.bitwise_xor(k, _ALLONES))


def _sc_sort_body(xs_hbm, out_hbm, buf_f, key_a, key_b, hist):
    wid = lax.axis_index("s") * NC + lax.axis_index("c")
    lanes = lax.iota(jnp.int32, L)
    ones = jnp.full((L,), 1, jnp.int32)

    def do_row(rr, _carry):
        row = wid * RPW + rr
        pltpu.sync_copy(xs_hbm.at[row], buf_f)

        for p in range(4):
            src, dst = [(buf_f, key_a), (key_a, key_b),
                        (key_b, key_a), (key_a, buf_f)][p]

            def zero_body(j, _, hist=hist):
                hist[pl.ds(j * L, L)] = jnp.zeros((L,), jnp.int32)
                return 0
            lax.fori_loop(0, BINS, zero_body, 0)

            def read_key(i, src=src, p=p):
                v = src[pl.ds(i * L, L)]
                return _to_key(v) if p == 0 else v

            def hist_body(i, _, p=p):
                k = read_key(i)
                idx = _digit(k, p) * L + lanes
                plsc.addupdate_scatter(hist, [idx], ones)
                return 0
            lax.fori_loop(0, C, hist_body, 0)

            def scan_body(j, carry, hist=hist):
                v = hist[pl.ds(j * L, L)]
                incl = plsc.cumsum(v)
                hist[pl.ds(j * L, L)] = incl - v + carry
                return carry + jnp.sum(v)
            lax.fori_loop(0, BINS, scan_body, jnp.int32(0))

            def perm_body(i, _, p=p, dst=dst):
                k = read_key(i)
                idx = _digit(k, p) * L + lanes
                r = plsc.load_gather(hist, [idx])
                plsc.store_scatter(hist, [idx], r + ones)
                if p < 3:
                    pos = jnp.bitwise_or(
                        lax.shift_left(jnp.bitwise_and(
                            r, jnp.full((L,), C - 1, jnp.int32)),
                            jnp.full((L,), 4, jnp.int32)),
                        lax.shift_right_logical(
                            r, jnp.full((L,), 11, jnp.int32)))
                    plsc.store_scatter(dst, [pos], k)
                else:
                    plsc.store_scatter(dst, [r], _from_key(k))
                return 0
            lax.fori_loop(0, C, perm_body, 0)

        pltpu.sync_copy(buf_f, out_hbm.at[row])
        return 0

    lax.fori_loop(0, RPW, do_row, 0)


_sc_sort = pl.kernel(
    _sc_sort_body,
    out_type=jax.ShapeDtypeStruct((ROWS, N), jnp.int32),
    mesh=plsc.VectorSubcoreMesh(core_axis_name="c", subcore_axis_name="s"),
    compiler_params=pltpu.CompilerParams(needs_layout_passes=False),
    scratch_types=[
        pltpu.VMEM((N,), jnp.int32),
        pltpu.VMEM((N,), jnp.int32),
        pltpu.VMEM((N,), jnp.int32),
        pltpu.VMEM((BINS * L,), jnp.int32),
    ],
)


def kernel(xs):
    xs_i = lax.bitcast_convert_type(xs, jnp.int32)
    return lax.bitcast_convert_type(_sc_sort(xs_i), jnp.float32)


# speedup vs baseline: 2.5515x; 1.1211x over previous
import jax
import jax.numpy as jnp
from jax import lax
from jax.experimental import pallas as pl
from jax.experimental.pallas import tpu as pltpu
from jax.experimental.pallas import tpu_sc as plsc

ROWS = 128
N = 32768
L = 16
C = N // L
BINS = 256
NC, NS = 2, 16
NW = NC * NS
RPW = ROWS // NW

import numpy as np

_MININT = np.int32(-(2 ** 31))
_ALLONES = np.int32(-1)


def _digit(k, p):
    if p == 0:
        s = k
    else:
        s = lax.shift_right_logical(k, jnp.full((L,), 8 * p, jnp.int32))
    return jnp.bitwise_and(s, jnp.full((L,), 255, jnp.int32))


def _to_key(b):
    return jnp.where(b < 0, jnp.bitwise_xor(b, _ALLONES),
                     jnp.bitwise_xor(b, _MININT))


def _from_key(k):
    return jnp.where(k < 0, jnp.bitwise_xor(k, _MININT),
                     jnp.bitwise_xor(k, _ALLONES))


def _sc_sort_body(xs_hbm, out_hbm, buf_f, key_a, key_b, hist):
    wid = lax.axis_index("s") * NC + lax.axis_index("c")
    lanes = lax.iota(jnp.int32, L)
    ones = jnp.full((L,), 1, jnp.int32)

    def do_row(rr, _carry):
        row = wid * RPW + rr
        pltpu.sync_copy(xs_hbm.at[row], buf_f)

        for p in range(4):
            src, dst = [(buf_f, key_a), (key_a, key_b),
                        (key_b, key_a), (key_a, buf_f)][p]

            def zero_body(j, _, hist=hist):
                hist[pl.ds(j * L, L)] = jnp.zeros((L,), jnp.int32)
                return 0
            lax.fori_loop(0, BINS, zero_body, 0, unroll=8)

            def read_key(i, src=src, p=p):
                v = src[pl.ds(i * L, L)]
                return _to_key(v) if p == 0 else v

            def hist_body(i, _, p=p):
                k = read_key(i)
                idx = _digit(k, p) * L + lanes
                plsc.addupdate_scatter(hist, [idx], ones)
                return 0
            lax.fori_loop(0, C, hist_body, 0, unroll=8)

            def scan_body(j, carry, hist=hist):
                v = hist[pl.ds(j * L, L)]
                incl = plsc.cumsum(v)
                hist[pl.ds(j * L, L)] = incl - v + carry
                return carry + jnp.sum(v)
            lax.fori_loop(0, BINS, scan_body, jnp.int32(0), unroll=4)

            def perm_body(i, _, p=p, dst=dst):
                k = read_key(i)
                idx = _digit(k, p) * L + lanes
                r = plsc.load_gather(hist, [idx])
                plsc.store_scatter(hist, [idx], r + ones)
                if p < 3:
                    pos = jnp.bitwise_or(
                        lax.shift_left(jnp.bitwise_and(
                            r, jnp.full((L,), C - 1, jnp.int32)),
                            jnp.full((L,), 4, jnp.int32)),
                        lax.shift_right_logical(
                            r, jnp.full((L,), 11, jnp.int32)))
                    plsc.store_scatter(dst, [pos], k)
                else:
                    plsc.store_scatter(dst, [r], _from_key(k))
                return 0
            lax.fori_loop(0, C, perm_body, 0, unroll=4)

        pltpu.sync_copy(buf_f, out_hbm.at[row])
        return 0

    lax.fori_loop(0, RPW, do_row, 0)


_sc_sort = pl.kernel(
    _sc_sort_body,
    out_type=jax.ShapeDtypeStruct((ROWS, N), jnp.int32),
    mesh=plsc.VectorSubcoreMesh(core_axis_name="c", subcore_axis_name="s"),
    compiler_params=pltpu.CompilerParams(needs_layout_passes=False),
    scratch_types=[
        pltpu.VMEM((N,), jnp.int32),
        pltpu.VMEM((N,), jnp.int32),
        pltpu.VMEM((N,), jnp.int32),
        pltpu.VMEM((BINS * L,), jnp.int32),
    ],
)


def kernel(xs):
    xs_i = lax.bitcast_convert_type(xs, jnp.int32)
    return lax.bitcast_convert_type(_sc_sort(xs_i), jnp.float32)
